# baseline (speedup 1.0000x reference)
import functools

import jax
import jax.numpy as jnp
from jax import lax
from jax.experimental import pallas as pl
from jax.experimental.pallas import tpu as pltpu
from jax.experimental.pallas import tpu_sc as plsc

NW = 32
BW = 128
L = 16


def _make_pack_transpose(V, D, H, W=4096):
  assert H % W == 0
  grid = H // W
  shift = H // W
  last = (V - 1) // W

  @functools.partial(
      pl.pallas_call,
      grid=(grid,),
      in_specs=[
          pl.BlockSpec((D, W), lambda i: (0, i)),
          pl.BlockSpec((D, W), lambda i: (0, jnp.minimum(i + shift, last))),
      ],
      out_specs=pl.BlockSpec((W, 2 * D), lambda i: (i, 0)),
      out_shape=jax.ShapeDtypeStruct((H, 2 * D), jnp.float32),
  )
  def body(x1_ref, x2_ref, o_ref):
    eye = jnp.eye(D, dtype=jnp.float32)

    def t(x):
      return lax.dot_general(
          x, eye, (((0,), (0,)), ((), ())),
          preferred_element_type=jnp.float32,
          precision=lax.Precision.HIGHEST,
      )

    o_ref[...] = jnp.concatenate([t(x1_ref[...]), t(x2_ref[...])], axis=1)

  return body


def _make_gather(S, B0, H, D):
  mesh = plsc.VectorSubcoreMesh(core_axis_name="c", subcore_axis_name="s")
  assert S % 2 == 0 and B0 == NW * BW

  @functools.partial(
      pl.kernel,
      mesh=mesh,
      out_type=jax.ShapeDtypeStruct((S, D, B0), jnp.float32),
      compiler_params=pltpu.CompilerParams(needs_layout_passes=False),
      scratch_types=[
          pltpu.VMEM((S, BW), jnp.int32),
          pltpu.VMEM((S, BW), jnp.int32),
          pltpu.VMEM((2, BW, 2 * D), jnp.float32),
          pltpu.VMEM((2, D, BW), jnp.float32),
          pltpu.SemaphoreType.DMA,
          pltpu.SemaphoreType.DMA,
          pltpu.SemaphoreType.DMA,
          pltpu.SemaphoreType.DMA,
      ],
  )
  def body(idx_hbm, table_hbm, out_hbm, idx_v, idx_h, buf_a, buf_b,
           g0, g1, o0, o1):
    wid = lax.axis_index("s") * 2 + lax.axis_index("c")
    b0 = wid * BW
    gsem = (g0, g1)
    osem = (o0, o1)

    pltpu.sync_copy(idx_hbm.at[:, pl.ds(b0, BW)], idx_v)

    def halve(s, carry):
      for g in range(BW // L):
        sl = pl.ds(g * L, L)
        v = idx_v[s, sl]
        idx_h[s, sl] = jnp.where(v >= H, v - H, v)
      return carry

    lax.fori_loop(0, S, halve, 0)

    def gather(s, p):
      return pltpu.make_async_copy(
          table_hbm.at[idx_h.at[s]], buf_a.at[p], gsem[p])

    def store(s, p):
      return pltpu.make_async_copy(
          buf_b.at[p], out_hbm.at[s].at[:, pl.ds(b0, BW)], osem[p])

    lane = lax.iota(jnp.int32, L)
    row_ids = [lane + g * L for g in range(BW // L)]

    def transpose(s, p):
      src = buf_a.at[p]
      dst = buf_b.at[p]
      half = [
          jnp.where(idx_v[s, pl.ds(g * L, L)] >= H,
                    jnp.int32(D), jnp.int32(0))
          for g in range(BW // L)
      ]

      def step(dd, carry):
        for q in range(4):
          diag = (lane + dd * 4 + q) & (D - 1)
          for g in range(BW // L):
            vals = plsc.load_gather(src, [row_ids[g], diag + half[g]])
            plsc.store_scatter(dst, [diag, row_ids[g]], vals)
        return carry

      lax.fori_loop(0, D // 4, step, 0)

    gather(0, 0).start()

    def half_iter(s, p):
      np_ = 1 - p

      @pl.when(s + 1 < S)
      def _():
        gather(s + 1, np_).start()

      gather(s, p).wait()

      @pl.when(s >= 2)
      def _():
        store(s - 2, p).wait()

      transpose(s, p)
      store(s, p).start()

    def pair(k, carry):
      half_iter(2 * k, 0)
      half_iter(2 * k + 1, 1)
      return carry

    lax.fori_loop(0, S // 2, pair, 0)
    store(S - 2, 0).wait()
    store(S - 1, 1).wait()

  return body


def kernel(inputs, weight):
  B0, S = inputs.shape
  V, D = weight.shape
  idx_t = inputs.T.astype(jnp.int32)
  wt = weight.T
  H = 512000
  table = _make_pack_transpose(V, D, H)(wt, wt)
  out_t = _make_gather(S, B0, H, D)(idx_t, table)
  return out_t.transpose(2, 0, 1)


# speedup vs baseline: 1.3047x; 1.3047x over previous
import functools

import jax
import jax.numpy as jnp
from jax import lax
from jax.experimental import pallas as pl
from jax.experimental.pallas import tpu as pltpu
from jax.experimental.pallas import tpu_sc as plsc

NW = 32
BW = 128
L = 16


def _make_pack_transpose(V, D, H, W=8192):
  assert H % W == 0
  grid = H // W
  shift = H // W
  last = (V - 1) // W

  @functools.partial(
      pl.pallas_call,
      grid=(grid,),
      in_specs=[
          pl.BlockSpec((D, W), lambda i: (0, i)),
          pl.BlockSpec((D, W), lambda i: (0, jnp.minimum(i + shift, last))),
      ],
      out_specs=pl.BlockSpec((W, 2 * D), lambda i: (i, 0)),
      out_shape=jax.ShapeDtypeStruct((H, 2 * D), jnp.float32),
  )
  def body(x1_ref, x2_ref, o_ref):
    eye = jnp.eye(D, dtype=jnp.bfloat16)

    def t(x):
      h1 = x.astype(jnp.bfloat16)
      r1 = x - h1.astype(jnp.float32)
      h2 = r1.astype(jnp.bfloat16)
      h3 = (r1 - h2.astype(jnp.float32)).astype(jnp.bfloat16)
      dn = (((0,), (0,)), ((), ()))
      acc = lax.dot_general(h1, eye, dn,
                            preferred_element_type=jnp.float32)
      acc += lax.dot_general(h2, eye, dn,
                             preferred_element_type=jnp.float32)
      acc += lax.dot_general(h3, eye, dn,
                             preferred_element_type=jnp.float32)
      return acc

    o_ref[...] = jnp.concatenate([t(x1_ref[...]), t(x2_ref[...])], axis=1)

  return body


def _make_gather(S, B0, H, D):
  mesh = plsc.VectorSubcoreMesh(core_axis_name="c", subcore_axis_name="s")
  assert S % 2 == 0 and B0 == NW * BW

  @functools.partial(
      pl.kernel,
      mesh=mesh,
      out_type=jax.ShapeDtypeStruct((S, D, B0), jnp.float32),
      compiler_params=pltpu.CompilerParams(needs_layout_passes=False),
      scratch_types=[
          pltpu.VMEM((S, BW), jnp.int32),
          pltpu.VMEM((S, BW), jnp.int32),
          pltpu.VMEM((2, BW, 2 * D), jnp.float32),
          pltpu.VMEM((2, D, BW), jnp.float32),
          pltpu.SemaphoreType.DMA,
          pltpu.SemaphoreType.DMA,
          pltpu.SemaphoreType.DMA,
          pltpu.SemaphoreType.DMA,
      ],
  )
  def body(idx_hbm, table_hbm, out_hbm, idx_v, idx_h, buf_a, buf_b,
           g0, g1, o0, o1):
    wid = lax.axis_index("s") * 2 + lax.axis_index("c")
    b0 = wid * BW
    gsem = (g0, g1)
    osem = (o0, o1)

    pltpu.sync_copy(idx_hbm.at[:, pl.ds(b0, BW)], idx_v)

    def halve(s, carry):
      for g in range(BW // L):
        sl = pl.ds(g * L, L)
        v = idx_v[s, sl]
        idx_h[s, sl] = jnp.where(v >= H, v - H, v)
      return carry

    lax.fori_loop(0, S, halve, 0)

    def gather(s, p):
      return pltpu.make_async_copy(
          table_hbm.at[idx_h.at[s]], buf_a.at[p], gsem[p])

    def store(s, p):
      return pltpu.make_async_copy(
          buf_b.at[p], out_hbm.at[s].at[:, pl.ds(b0, BW)], osem[p])

    lane = lax.iota(jnp.int32, L)
    row_ids = [lane + g * L for g in range(BW // L)]

    def transpose(s, p):
      src = buf_a.at[p]
      dst = buf_b.at[p]
      half = [
          jnp.where(idx_v[s, pl.ds(g * L, L)] >= H,
                    jnp.int32(D), jnp.int32(0))
          for g in range(BW // L)
      ]

      def step(dd, carry):
        for q in range(4):
          diag = (lane + dd * 4 + q) & (D - 1)
          for g in range(BW // L):
            vals = plsc.load_gather(src, [row_ids[g], diag + half[g]])
            plsc.store_scatter(dst, [diag, row_ids[g]], vals)
        return carry

      lax.fori_loop(0, D // 4, step, 0)

    gather(0, 0).start()

    def half_iter(s, p):
      np_ = 1 - p

      @pl.when(s + 1 < S)
      def _():
        gather(s + 1, np_).start()

      gather(s, p).wait()

      @pl.when(s >= 2)
      def _():
        store(s - 2, p).wait()

      transpose(s, p)
      store(s, p).start()

    def pair(k, carry):
      half_iter(2 * k, 0)
      half_iter(2 * k + 1, 1)
      return carry

    lax.fori_loop(0, S // 2, pair, 0)
    store(S - 2, 0).wait()
    store(S - 1, 1).wait()

  return body


def kernel(inputs, weight):
  B0, S = inputs.shape
  V, D = weight.shape
  idx_t = inputs.T.astype(jnp.int32)
  wt = weight.T
  H = 516096
  table = _make_pack_transpose(V, D, H)(wt, wt)
  out_t = _make_gather(S, B0, H, D)(idx_t, table)
  return out_t.transpose(2, 0, 1)


# speedup vs baseline: 1.3236x; 1.0145x over previous
import functools

import jax
import jax.numpy as jnp
from jax import lax
from jax.experimental import pallas as pl
from jax.experimental.pallas import tpu as pltpu
from jax.experimental.pallas import tpu_sc as plsc

NW = 32
BW = 128
L = 16


def _make_pack_transpose(V, D, H, W=8192):
  assert H % W == 0
  grid = H // W
  shift = H // W
  last = (V - 1) // W

  @functools.partial(
      pl.pallas_call,
      grid=(grid,),
      in_specs=[
          pl.BlockSpec((D, W), lambda i: (0, i)),
          pl.BlockSpec((D, W), lambda i: (0, jnp.minimum(i + shift, last))),
      ],
      out_specs=pl.BlockSpec((W, 2 * D), lambda i: (i, 0)),
      out_shape=jax.ShapeDtypeStruct((H, 2 * D), jnp.float32),
  )
  def body(x1_ref, x2_ref, o_ref):
    eye = jnp.eye(D, dtype=jnp.bfloat16)

    def t(x):
      h1 = x.astype(jnp.bfloat16)
      r1 = x - h1.astype(jnp.float32)
      h2 = r1.astype(jnp.bfloat16)
      h3 = (r1 - h2.astype(jnp.float32)).astype(jnp.bfloat16)
      dn = (((0,), (0,)), ((), ()))
      acc = lax.dot_general(h1, eye, dn,
                            preferred_element_type=jnp.float32)
      acc += lax.dot_general(h2, eye, dn,
                             preferred_element_type=jnp.float32)
      acc += lax.dot_general(h3, eye, dn,
                             preferred_element_type=jnp.float32)
      return acc

    o_ref[...] = jnp.concatenate([t(x1_ref[...]), t(x2_ref[...])], axis=1)

  return body


def _make_gather(S, B0, H, D):
  mesh = plsc.VectorSubcoreMesh(core_axis_name="c", subcore_axis_name="s")
  assert S % 2 == 0 and B0 == NW * BW

  @functools.partial(
      pl.kernel,
      mesh=mesh,
      out_type=jax.ShapeDtypeStruct((S, D, B0), jnp.float32),
      compiler_params=pltpu.CompilerParams(needs_layout_passes=False),
      scratch_types=[
          pltpu.VMEM((S, BW), jnp.int32),
          pltpu.VMEM((S, BW), jnp.int32),
          pltpu.VMEM((2, BW, 2 * D), jnp.float32),
          pltpu.VMEM((2, D, BW), jnp.float32),
          pltpu.SemaphoreType.DMA,
          pltpu.SemaphoreType.DMA,
          pltpu.SemaphoreType.DMA,
          pltpu.SemaphoreType.DMA,
      ],
  )
  def body(idx_hbm, table_hbm, out_hbm, idx_v, idx_h, buf_a, buf_b,
           g0, g1, o0, o1):
    wid = lax.axis_index("s") * 2 + lax.axis_index("c")
    b0 = wid * BW
    gsem = (g0, g1)
    osem = (o0, o1)

    pltpu.sync_copy(idx_hbm.at[:, pl.ds(b0, BW)], idx_v)

    def halve(s, carry):
      for g in range(BW // L):
        sl = pl.ds(g * L, L)
        v = idx_v[s, sl]
        idx_h[s, sl] = jnp.where(v >= H, v - H, v)
      return carry

    lax.fori_loop(0, S, halve, 0)

    def gather(s, p):
      return pltpu.make_async_copy(
          table_hbm.at[idx_h.at[s]], buf_a.at[p], gsem[p])

    def store(s, p):
      return pltpu.make_async_copy(
          buf_b.at[p], out_hbm.at[s].at[:, pl.ds(b0, BW)], osem[p])

    lane = lax.iota(jnp.int32, L)
    row_ids = [lane + g * L for g in range(BW // L)]

    def transpose(s, p):
      src = buf_a.at[p]
      dst = buf_b.at[p]
      half = [
          jnp.where(idx_v[s, pl.ds(g * L, L)] >= H,
                    jnp.int32(D), jnp.int32(0))
          for g in range(BW // L)
      ]

      def step(dd, carry):
        for q in range(8):
          diag = (lane + dd * 8 + q) & (D - 1)
          for g in range(BW // L):
            vals = plsc.load_gather(src, [row_ids[g], diag + half[g]])
            plsc.store_scatter(dst, [diag, row_ids[g]], vals)
        return carry

      lax.fori_loop(0, D // 8, step, 0)

    gather(0, 0).start()

    def half_iter(s, p):
      np_ = 1 - p

      @pl.when(s + 1 < S)
      def _():
        gather(s + 1, np_).start()

      gather(s, p).wait()

      @pl.when(s >= 2)
      def _():
        store(s - 2, p).wait()

      transpose(s, p)
      store(s, p).start()

    def pair(k, carry):
      half_iter(2 * k, 0)
      half_iter(2 * k + 1, 1)
      return carry

    lax.fori_loop(0, S // 2, pair, 0)
    store(S - 2, 0).wait()
    store(S - 1, 1).wait()

  return body


def kernel(inputs, weight):
  B0, S = inputs.shape
  V, D = weight.shape
  idx_t = inputs.T.astype(jnp.int32)
  wt = weight.T
  H = 516096
  table = _make_pack_transpose(V, D, H)(wt, wt)
  out_t = _make_gather(S, B0, H, D)(idx_t, table)
  return out_t.transpose(2, 0, 1)
